# baseline (speedup 1.0000x reference)
import functools

import jax
import jax.numpy as jnp
from jax import lax
from jax.experimental import pallas as pl
from jax.experimental.pallas import tpu as pltpu
from jax.experimental.pallas import tpu_sc as plsc

B, T, V = 1024, 50, 1000
N = B * T
NC, NS, L = 2, 16, 16
NW = NC * NS
BPW = B // NW
R = N // NW
TCH = 4


def _nll_body(inp_hbm, tgt_hbm, msk_hbm, loss_out, mask_out,
              buf_v, tgt_v, msk_v, stage_v, sem0, sem1):
    cid = lax.axis_index("c")
    sid = lax.axis_index("s")
    wid = cid * NS + sid
    base = wid * R
    b0 = wid * BPW

    pltpu.sync_copy(tgt_hbm.at[pl.ds(base, R)], tgt_v)
    pltpu.sync_copy(msk_hbm.at[pl.ds(base, R)], msk_v)

    lane = lax.iota(jnp.int32, L)

    acc_l = jnp.zeros((L,), jnp.float32)
    acc_m = jnp.zeros((L,), jnp.float32)
    for blk in range(BPW // 2):
        pltpu.make_async_copy(
            inp_hbm.at[pl.ds(b0 + 2 * blk, 2)], buf_v, sem0).start()
        pltpu.make_async_copy(
            inp_hbm.at[pl.ds(b0 + 2 * blk, 2)], buf_v, sem0).wait()
        for bl in (2 * blk, 2 * blk + 1):
            cur = buf_v.at[bl & 1]
            for j in range(TCH):
                rows = j * L + lane
                valid = rows < T
                t_idx = jnp.minimum(rows, T - 1)
                gof = bl * T + t_idx
                tv = plsc.load_gather(tgt_v, [gof])
                tv = jnp.maximum(tv - 1, 0)
                e = plsc.load_gather(cur, [t_idx, tv])
                m = plsc.load_gather(msk_v, [gof])
                m = jnp.where(valid, m, 0.0)
                acc_l = acc_l + e * m
                acc_m = acc_m + m

    stage_v[...] = acc_l
    pltpu.sync_copy(stage_v, loss_out.at[pl.ds(wid * L, L)])
    stage_v[...] = acc_m
    pltpu.sync_copy(stage_v, mask_out.at[pl.ds(wid * L, L)])


_nll_kernel = functools.partial(
    pl.kernel,
    out_type=[
        jax.ShapeDtypeStruct((NW * L,), jnp.float32),
        jax.ShapeDtypeStruct((NW * L,), jnp.float32),
    ],
    mesh=plsc.VectorSubcoreMesh(core_axis_name="c", subcore_axis_name="s"),
    compiler_params=pltpu.CompilerParams(needs_layout_passes=False),
    scratch_types=[
        pltpu.VMEM((2, T, V), jnp.float32),
        pltpu.VMEM((R,), jnp.int32),
        pltpu.VMEM((R,), jnp.float32),
        pltpu.VMEM((L,), jnp.float32),
        pltpu.SemaphoreType.DMA,
        pltpu.SemaphoreType.DMA,
    ],
)(_nll_body)


@jax.jit
def kernel(input, target, mask):
    Tt = input.shape[1]
    tgt = target[:, :Tt].reshape(-1).astype(jnp.int32)
    msk = mask[:, :Tt].reshape(-1).astype(jnp.float32)
    loss_p, mask_p = _nll_kernel(input, tgt, msk)
    return -jnp.sum(loss_p) / jnp.sum(mask_p)
